# baseline (device time: 67483 ns/iter reference)
import os

import jax
import jax.numpy as jnp
from jax import lax
from jax.experimental import pallas as pl
from jax.experimental.pallas import tpu as pltpu

N_DEV = 32


def kernel(x, Win0, Wout0, Win1, Wout1, Win2, Wout2):
    b, d_shard = x.shape
    h_dim = Win0.shape[1]
    ch = b // N_DEV
    assert ch * N_DEV == b

    def body(x_ref, win0_ref, wout0_ref, win1_ref, wout1_ref, win2_ref,
             wout2_ref, out_ref, h_ref, rs_ref, hfull_ref, red_ref,
             rs_send, rs_recv, ag_send, ag_recv, local_sem):
        me = lax.axis_index("i")

        barrier = pltpu.get_barrier_semaphore()
        for t in range(N_DEV):
            @pl.when(t != me)
            def _():
                pl.semaphore_signal(barrier, inc=1, device_id=(t,),
                                    device_id_type=pl.DeviceIdType.MESH)
        pl.semaphore_wait(barrier, N_DEV - 1)

        x_val = x_ref[...]
        layers = ((win0_ref, wout0_ref), (win1_ref, wout1_ref),
                  (win2_ref, wout2_ref))
        for win_ref, wout_ref in layers:
            partial = jnp.dot(x_val.astype(jnp.bfloat16),
                              win_ref[...].astype(jnp.bfloat16),
                              preferred_element_type=jnp.float32)
            h_ref[...] = partial.astype(jnp.bfloat16)

            own = pltpu.make_async_copy(
                h_ref.at[pl.ds(ch * me, ch), :],
                rs_ref.at[pl.ds(ch * me, ch), :],
                local_sem,
            )
            own.start()
            rs_sends = []
            for t in range(N_DEV):
                rdma = pltpu.make_async_remote_copy(
                    src_ref=h_ref.at[pl.ds(ch * t, ch), :],
                    dst_ref=rs_ref.at[pl.ds(ch * me, ch), :],
                    send_sem=rs_send.at[t],
                    recv_sem=rs_recv.at[me],
                    device_id=(t,),
                    device_id_type=pl.DeviceIdType.MESH,
                )

                @pl.when(t != me)
                def _(rdma=rdma):
                    rdma.start()

                rs_sends.append(rdma)
            for k in range(N_DEV):
                recv = pltpu.make_async_remote_copy(
                    src_ref=rs_ref.at[pl.ds(ch * k, ch), :],
                    dst_ref=rs_ref.at[pl.ds(ch * k, ch), :],
                    send_sem=rs_send.at[k],
                    recv_sem=rs_recv.at[k],
                    device_id=(k,),
                    device_id_type=pl.DeviceIdType.MESH,
                )

                @pl.when(k != me)
                def _(recv=recv):
                    recv.wait_recv()
            for t, rdma in enumerate(rs_sends):
                @pl.when(t != me)
                def _(rdma=rdma):
                    rdma.wait_send()
            own.wait()

            rs_val = rs_ref[...].astype(jnp.float32)
            acc = rs_val[0:ch, :]
            for k in range(1, N_DEV):
                acc = acc + rs_val[ch * k:ch * (k + 1), :]
            red_ref[...] = jnp.maximum(acc, 0.0).astype(jnp.bfloat16)
            mine = pltpu.make_async_copy(
                red_ref,
                hfull_ref.at[pl.ds(ch * me, ch), :],
                local_sem,
            )
            mine.start()

            ag_sends = []
            for t in range(N_DEV):
                rdma = pltpu.make_async_remote_copy(
                    src_ref=red_ref,
                    dst_ref=hfull_ref.at[pl.ds(ch * me, ch), :],
                    send_sem=ag_send.at[t],
                    recv_sem=ag_recv.at[me],
                    device_id=(t,),
                    device_id_type=pl.DeviceIdType.MESH,
                )

                @pl.when(t != me)
                def _(rdma=rdma):
                    rdma.start()

                ag_sends.append(rdma)
            for k in range(N_DEV):
                recv = pltpu.make_async_remote_copy(
                    src_ref=red_ref,
                    dst_ref=hfull_ref.at[pl.ds(ch * k, ch), :],
                    send_sem=ag_send.at[k],
                    recv_sem=ag_recv.at[k],
                    device_id=(k,),
                    device_id_type=pl.DeviceIdType.MESH,
                )

                @pl.when(k != me)
                def _(recv=recv):
                    recv.wait_recv()
            for t, rdma in enumerate(ag_sends):
                @pl.when(t != me)
                def _(rdma=rdma):
                    rdma.wait_send()
            mine.wait()

            x_val = jnp.dot(hfull_ref[...],
                            wout_ref[...].astype(jnp.bfloat16),
                            preferred_element_type=jnp.float32)

        out_ref[...] = x_val

    interpret = (pltpu.InterpretParams()
                 if os.environ.get("SCBAND_INTERPRET") else False)
    return pl.pallas_call(
        body,
        out_shape=jax.ShapeDtypeStruct((b, d_shard), jnp.float32),
        in_specs=[pl.BlockSpec(memory_space=pltpu.VMEM)] * 7,
        out_specs=pl.BlockSpec(memory_space=pltpu.VMEM),
        scratch_shapes=[
            pltpu.VMEM((b, h_dim), jnp.bfloat16),
            pltpu.VMEM((b, h_dim), jnp.bfloat16),
            pltpu.VMEM((b, h_dim), jnp.bfloat16),
            pltpu.VMEM((ch, h_dim), jnp.bfloat16),
            pltpu.SemaphoreType.DMA((N_DEV,)),
            pltpu.SemaphoreType.DMA((N_DEV,)),
            pltpu.SemaphoreType.DMA((N_DEV,)),
            pltpu.SemaphoreType.DMA((N_DEV,)),
            pltpu.SemaphoreType.DMA,
        ],
        compiler_params=pltpu.CompilerParams(
            collective_id=0, vmem_limit_bytes=60 * 1024 * 1024),
        interpret=interpret,
    )(x, Win0, Wout0, Win1, Wout1, Win2, Wout2)
